# baseline (speedup 1.0000x reference)
import functools

import jax
import jax.numpy as jnp
from jax import lax
from jax.experimental import pallas as pl
from jax.experimental.pallas import tpu as pltpu
from jax.experimental.pallas import tpu_sc as plsc

N_TOKENS = 16384
N_EXP = 16
_NC, _NS = 2, 16
_NW = _NC * _NS
_ROWS_PER_W = N_TOKENS // _NW
_BLOCKS = _ROWS_PER_W // 16


_mesh = plsc.VectorSubcoreMesh(core_axis_name="c", subcore_axis_name="s")


@functools.partial(
    pl.kernel,
    mesh=_mesh,
    compiler_params=pltpu.CompilerParams(
        needs_layout_passes=False, use_tc_tiling_on_sc=False),
    out_type=[
        jax.ShapeDtypeStruct((_NW, N_EXP, 16), jnp.float32),
        jax.ShapeDtypeStruct((_NW, N_EXP, 16), jnp.float32),
    ],
    scratch_types=[
        pltpu.VMEM((_ROWS_PER_W, N_EXP), jnp.float32),
        pltpu.VMEM((N_EXP, 16), jnp.float32),
        pltpu.VMEM((N_EXP, 16), jnp.float32),
    ],
)
def _sc_tally(logits_hbm, pacc_hbm, cacc_hbm, buf, pacc, cacc):
    wid = lax.axis_index("s") * _NC + lax.axis_index("c")
    row0 = wid * _ROWS_PER_W
    pltpu.sync_copy(logits_hbm.at[pl.ds(row0, _ROWS_PER_W), :], buf)

    zero = jnp.zeros((16,), jnp.float32)
    for e in range(N_EXP):
        pacc[e] = zero
        cacc[e] = zero

    row_iota = lax.iota(jnp.int32, 16)

    def body(b, carry):
        rows = row_iota + b * 16
        xs = [plsc.load_gather(buf, [rows, row_iota * 0 + e])
              for e in range(N_EXP)]
        m = xs[0]
        for e in range(1, N_EXP):
            m = jnp.maximum(m, xs[e])
        ts = [jnp.exp(x - m) for x in xs]
        s = ts[0]
        for e in range(1, N_EXP):
            s = s + ts[e]
        r = jnp.float32(1.0) / s
        seen = jnp.zeros((16,), jnp.bool_)
        one = jnp.ones((16,), jnp.float32)
        zf = jnp.zeros((16,), jnp.float32)
        for e in range(N_EXP):
            pacc[e] = pacc[e] + ts[e] * r
            eq = xs[e] == m
            first = jnp.logical_and(eq, jnp.logical_not(seen))
            seen = jnp.logical_or(seen, eq)
            cacc[e] = cacc[e] + jnp.where(first, one, zf)
        return carry

    lax.fori_loop(0, _BLOCKS, body, 0)
    pltpu.sync_copy(pacc, pacc_hbm.at[wid])
    pltpu.sync_copy(cacc, cacc_hbm.at[wid])


def _tc_finish(p_ref, c_ref, w_ref, o_ref):
    S = jnp.sum(p_ref[...], axis=(0, 2))
    C = jnp.sum(c_ref[...], axis=(0, 2))
    w = jnp.abs(w_ref[0, 0])
    pbar = S / jnp.float32(N_TOKENS)
    f = C / jnp.sum(C)
    o_ref[0, 0] = w * jnp.float32(N_EXP) * jnp.sum(pbar * f)


@jax.jit
def kernel(router_logits, wBAL):
    unit = jnp.exp(jnp.float32(0.0) * wBAL)
    pacc, cacc = _sc_tally(router_logits * unit)
    loss = pl.pallas_call(
        _tc_finish,
        out_shape=jax.ShapeDtypeStruct((1, 1), jnp.float32),
        in_specs=[
            pl.BlockSpec(memory_space=pltpu.VMEM),
            pl.BlockSpec(memory_space=pltpu.VMEM),
            pl.BlockSpec(memory_space=pltpu.SMEM),
        ],
        out_specs=pl.BlockSpec(memory_space=pltpu.SMEM),
    )(pacc, cacc, wBAL.reshape(1, 1))
    return loss[0, 0]


# speedup vs baseline: 1.3910x; 1.3910x over previous
import functools

import jax
import jax.numpy as jnp
from jax import lax
from jax.experimental import pallas as pl
from jax.experimental.pallas import tpu as pltpu
from jax.experimental.pallas import tpu_sc as plsc

N_TOKENS = 16384
N_EXP = 16
_NC, _NS = 2, 16
_NW = _NC * _NS
_ROWS_PER_W = N_TOKENS // _NW
_BLOCKS = _ROWS_PER_W // 16


_mesh = plsc.VectorSubcoreMesh(core_axis_name="c", subcore_axis_name="s")


_CHUNK_ROWS = 128
_N_CHUNKS = _ROWS_PER_W // _CHUNK_ROWS
_CHUNK_BLOCKS = _CHUNK_ROWS // 16


@functools.partial(
    pl.kernel,
    mesh=_mesh,
    compiler_params=pltpu.CompilerParams(needs_layout_passes=False),
    out_type=[
        jax.ShapeDtypeStruct((_NW, N_EXP, 16), jnp.float32),
        jax.ShapeDtypeStruct((_NW, N_EXP, 16), jnp.float32),
    ],
    scratch_types=[
        pltpu.VMEM((_CHUNK_ROWS, N_EXP), jnp.float32),
        pltpu.VMEM((_CHUNK_ROWS, N_EXP), jnp.float32),
        pltpu.VMEM((N_EXP, 16), jnp.float32),
        pltpu.VMEM((N_EXP, 16), jnp.float32),
        pltpu.SemaphoreType.DMA,
        pltpu.SemaphoreType.DMA,
    ],
)
def _sc_tally(logits_hbm, pacc_hbm, cacc_hbm, buf0, buf1, pacc, cacc,
              sem0, sem1):
    wid = lax.axis_index("s") * _NC + lax.axis_index("c")
    row0 = wid * _ROWS_PER_W
    bufs = (buf0, buf1)
    sems = (sem0, sem1)

    def chunk_src(k):
        return logits_hbm.at[pl.ds(row0 + k * _CHUNK_ROWS, _CHUNK_ROWS), :]

    copies = [pltpu.async_copy(chunk_src(k), bufs[k % 2], sems[k % 2])
              for k in range(2)]

    zero = jnp.zeros((16,), jnp.float32)
    for e in range(N_EXP):
        pacc[e] = zero
        cacc[e] = zero

    row_iota = lax.iota(jnp.int32, 16)
    cols = [row_iota * 0 + e for e in range(N_EXP)]
    one = jnp.ones((16,), jnp.float32)
    zf = jnp.zeros((16,), jnp.float32)
    false16 = jnp.zeros((16,), jnp.bool_)

    def make_body(buf):
        def body(b, carry):
            rows = row_iota + b * 16
            xs = [plsc.load_gather(buf, [rows, cols[e]]) for e in range(N_EXP)]
            m = xs[0]
            for e in range(1, N_EXP):
                m = jnp.maximum(m, xs[e])
            ts = [jnp.exp(x - m) for x in xs]
            s = ts[0]
            for e in range(1, N_EXP):
                s = s + ts[e]
            r = jnp.float32(1.0) / s
            seen = false16
            for e in range(N_EXP):
                pacc[e] = pacc[e] + ts[e] * r
                eq = xs[e] == m
                first = jnp.logical_and(eq, jnp.logical_not(seen))
                seen = jnp.logical_or(seen, eq)
                cacc[e] = cacc[e] + jnp.where(first, one, zf)
            return carry
        return body

    for k in range(_N_CHUNKS):
        copies[k].wait()
        lax.fori_loop(0, _CHUNK_BLOCKS, make_body(bufs[k % 2]), 0)
        if k + 2 < _N_CHUNKS:
            copies.append(
                pltpu.async_copy(chunk_src(k + 2), bufs[k % 2], sems[k % 2]))

    pltpu.sync_copy(pacc, pacc_hbm.at[wid])
    pltpu.sync_copy(cacc, cacc_hbm.at[wid])


def _tc_finish(p_ref, c_ref, w_ref, o_ref):
    S = jnp.sum(p_ref[...], axis=(0, 2))
    C = jnp.sum(c_ref[...], axis=(0, 2))
    w = jnp.abs(w_ref[0, 0])
    pbar = S / jnp.float32(N_TOKENS)
    f = C / jnp.sum(C)
    o_ref[0, 0] = w * jnp.float32(N_EXP) * jnp.sum(pbar * f)


@jax.jit
def kernel(router_logits, wBAL):
    pacc, cacc = _sc_tally(router_logits)
    loss = pl.pallas_call(
        _tc_finish,
        out_shape=jax.ShapeDtypeStruct((1, 1), jnp.float32),
        in_specs=[
            pl.BlockSpec(memory_space=pltpu.VMEM),
            pl.BlockSpec(memory_space=pltpu.VMEM),
            pl.BlockSpec(memory_space=pltpu.SMEM),
        ],
        out_specs=pl.BlockSpec(memory_space=pltpu.SMEM),
    )(pacc, cacc, wBAL.reshape(1, 1))
    return loss[0, 0]


# speedup vs baseline: 1.5074x; 1.0836x over previous
import functools

import jax
import jax.numpy as jnp
from jax import lax
from jax.experimental import pallas as pl
from jax.experimental.pallas import tpu as pltpu
from jax.experimental.pallas import tpu_sc as plsc

N_TOKENS = 16384
N_EXP = 16
_NC, _NS = 2, 16
_NW = _NC * _NS
_ROWS_PER_W = N_TOKENS // _NW
_BLOCKS = _ROWS_PER_W // 16


_mesh = plsc.VectorSubcoreMesh(core_axis_name="c", subcore_axis_name="s")


_CHUNK_ROWS = 128
_N_CHUNKS = _ROWS_PER_W // _CHUNK_ROWS
_CHUNK_BLOCKS = _CHUNK_ROWS // 16


@functools.partial(
    pl.kernel,
    mesh=_mesh,
    compiler_params=pltpu.CompilerParams(needs_layout_passes=False),
    out_type=[
        jax.ShapeDtypeStruct((_NW, N_EXP), jnp.float32),
        jax.ShapeDtypeStruct((_NW, N_EXP), jnp.float32),
    ],
    scratch_types=[
        pltpu.VMEM((_CHUNK_ROWS, N_EXP), jnp.float32),
        pltpu.VMEM((_CHUNK_ROWS, N_EXP), jnp.float32),
        pltpu.VMEM((N_EXP,), jnp.float32),
        pltpu.VMEM((N_EXP,), jnp.float32),
        pltpu.SemaphoreType.DMA,
        pltpu.SemaphoreType.DMA,
    ],
)
def _sc_tally(logits_hbm, pacc_hbm, cacc_hbm, buf0, buf1, pacc, cacc,
              sem0, sem1):
    wid = lax.axis_index("s") * _NC + lax.axis_index("c")
    row0 = wid * _ROWS_PER_W
    bufs = (buf0, buf1)
    sems = (sem0, sem1)

    def chunk_src(k):
        return logits_hbm.at[pl.ds(row0 + k * _CHUNK_ROWS, _CHUNK_ROWS), :]

    copies = [pltpu.async_copy(chunk_src(k), bufs[k % 2], sems[k % 2])
              for k in range(2)]

    row_iota = lax.iota(jnp.int32, 16)
    one = jnp.ones((16,), jnp.float32)
    zf = jnp.zeros((16,), jnp.float32)

    def make_body(buf):
        def body(b, carry):
            acc, cnt = carry
            x = buf[b]
            m = jnp.max(x)
            t = jnp.exp(x - m)
            s = jnp.sum(t)
            ff = plsc.all_reduce_ffs(x == m)
            acc = acc + t / s
            cnt = cnt + jnp.where(row_iota == ff, one, zf)
            return acc, cnt
        return body

    carry = (zf, zf)
    for k in range(_N_CHUNKS):
        copies[k].wait()
        carry = lax.fori_loop(0, _CHUNK_ROWS, make_body(bufs[k % 2]), carry)
        if k + 2 < _N_CHUNKS:
            copies.append(
                pltpu.async_copy(chunk_src(k + 2), bufs[k % 2], sems[k % 2]))

    pacc[...] = carry[0]
    cacc[...] = carry[1]
    pltpu.sync_copy(pacc, pacc_hbm.at[wid])
    pltpu.sync_copy(cacc, cacc_hbm.at[wid])


def _tc_finish(p_ref, c_ref, w_ref, o_ref):
    S = jnp.sum(p_ref[...], axis=0)
    C = jnp.sum(c_ref[...], axis=0)
    w = jnp.abs(w_ref[0, 0])
    pbar = S / jnp.float32(N_TOKENS)
    f = C / jnp.sum(C)
    o_ref[0, 0] = w * jnp.float32(N_EXP) * jnp.sum(pbar * f)


@jax.jit
def kernel(router_logits, wBAL):
    pacc, cacc = _sc_tally(router_logits)
    loss = pl.pallas_call(
        _tc_finish,
        out_shape=jax.ShapeDtypeStruct((1, 1), jnp.float32),
        in_specs=[
            pl.BlockSpec(memory_space=pltpu.VMEM),
            pl.BlockSpec(memory_space=pltpu.VMEM),
            pl.BlockSpec(memory_space=pltpu.SMEM),
        ],
        out_specs=pl.BlockSpec(memory_space=pltpu.SMEM),
    )(pacc, cacc, wBAL.reshape(1, 1))
    return loss[0, 0]
